# Initial kernel scaffold; baseline (speedup 1.0000x reference)
import jax
import jax.numpy as jnp
from jax.experimental import pallas as pl


def kernel(feat, feat_a, edge_index, graph_neigh, W1, a1_src, a1_dst, b1, W2, a2_src, a2_dst, b2, Wb, bb):
    raise NotImplementedError("write your pallas kernel here")


# speedup vs baseline: 26.0159x; 26.0159x over previous
import functools

import jax
import jax.numpy as jnp
from jax import lax
from jax.experimental import pallas as pl
from jax.experimental.pallas import tpu as pltpu
from jax.experimental.pallas import tpu_sc as plsc

F32 = jnp.float32
I32 = jnp.int32

NC = 2
NS = 16
NT = NC * NS
L = 16
K = 64
TW = 64
SW = TW + 16


def _make_sc_conv(NCHUNK, RPAD):
  RPT = RPAD // NS
  assert RPT % K == 0 and NCHUNK % 2 == 0

  mesh = plsc.VectorSubcoreMesh(core_axis_name="c", subcore_axis_name="s")

  scratch = [
      pltpu.VMEM((NCHUNK, K), I32),
      pltpu.VMEM((NCHUNK, K), I32),
      [pltpu.VMEM((RPAD,), F32) for _ in range(2)],
      pltpu.VMEM((16,), F32),
      pltpu.VMEM((K,), F32),
      [pltpu.VMEM((K, TW), F32) for _ in range(2)],
      [pltpu.VMEM((K, SW), F32) for _ in range(2)],
      pltpu.VMEM((K, SW), F32),
      [pltpu.SemaphoreType.DMA for _ in range(2)],
      [pltpu.SemaphoreType.DMA for _ in range(2)],
      pltpu.VMEM_SHARED((RPAD, SW), F32),
  ]

  @functools.partial(
      pl.kernel,
      out_type=jax.ShapeDtypeStruct((NC, RPAD, SW), F32),
      mesh=mesh,
      scratch_types=scratch,
      compiler_params=pltpu.CompilerParams(
          needs_layout_passes=False, use_tc_tiling_on_sc=False),
  )
  def conv(src_hbm, dst_hbm, table_hbm, st_hbm, cv_hbm, out_hbm,
           src_t, dst_t, st, cv, exb, rb, sb, zb, gsem, ssem, acc):
    c = lax.axis_index("c")
    s = lax.axis_index("s")
    tid = c * NS + s

    pltpu.sync_copy(src_hbm.at[tid], src_t)
    pltpu.sync_copy(dst_hbm.at[tid], dst_t)
    pltpu.sync_copy(st_hbm.at[0], st[0])
    pltpu.sync_copy(st_hbm.at[1], st[1])
    pltpu.sync_copy(cv_hbm, cv)

    zv = jnp.zeros((L,), F32)

    def zfill(j, carry):
      for kk in range(SW // L):
        zb[j, pl.ds(kk * L, L)] = zv
      return carry

    lax.fori_loop(0, K, zfill, 0)
    for kk in range(RPT // K):
      pltpu.sync_copy(zb, acc.at[pl.ds(s * RPT + kk * K, K)])
    plsc.subcore_barrier()

    onehot = jnp.where(lax.iota(I32, L) == 0, 1.0, 0.0).astype(F32)
    cvv = cv[:]

    def issue_gather(i, b):
      return pltpu.async_copy(table_hbm.at[src_t.at[i]], rb[b], gsem[b])

    def issue_scatter(i, b):
      return pltpu.async_copy(sb[b], acc.at[dst_t.at[i]], ssem[b], add=True)

    def wait_gather(i, b):
      pltpu.make_async_copy(table_hbm.at[src_t.at[i]], rb[b], gsem[b]).wait()

    def wait_scatter(i, b):
      pltpu.make_async_copy(sb[b], acc.at[dst_t.at[i]], ssem[b]).wait()

    def compute_ex(i):
      for kk in range(K // L):
        srcv = src_t[i, pl.ds(kk * L, L)]
        dstv = dst_t[i, pl.ds(kk * L, L)]
        a = plsc.load_gather(st[0], [srcv])
        bsc = plsc.load_gather(st[1], [dstv])
        e = a + bsc
        e = jnp.maximum(e, 0.2 * e)
        exb[pl.ds(kk * L, L)] = jnp.exp(e - cvv)

    def scale(b):
      def sbody(j, carry):
        jidx = jnp.full((L,), j, I32)
        exj = plsc.load_gather(exb, [jidx])
        for kk in range(TW // L):
          v = rb[b][j, pl.ds(kk * L, L)]
          sb[b][j, pl.ds(kk * L, L)] = v * exj
        sb[b][j, pl.ds(TW, L)] = exj * onehot
        return carry

      lax.fori_loop(0, K, sbody, 0)

    issue_gather(0, 0)

    def outer(it, carry):
      for b in range(2):
        i = it * 2 + b

        @pl.when(i + 1 < NCHUNK)
        def _():
          issue_gather(i + 1, 1 - b)

        compute_ex(i)
        wait_gather(i, b)

        @pl.when(i >= 2)
        def _():
          wait_scatter(i - 2, b)

        scale(b)
        issue_scatter(i, b)
      return carry

    lax.fori_loop(0, NCHUNK // 2, outer, 0)
    for b in range(2):
      wait_scatter(NCHUNK - 2 + b, b)

    plsc.subcore_barrier()
    pltpu.sync_copy(acc.at[pl.ds(s * RPT, RPT)],
                    out_hbm.at[c, pl.ds(s * RPT, RPT)])

  return conv


def _prep1_body(f_ref, fa_ref, w_ref, a_ref, xw_ref, xwa_ref, s_ref):
  xw1 = jnp.dot(f_ref[:], w_ref[:], preferred_element_type=F32)
  xw1a = jnp.dot(fa_ref[:], w_ref[:], preferred_element_type=F32)
  xw_ref[:] = xw1
  xwa_ref[:] = xw1a
  s_ref[:] = jnp.dot(jnp.concatenate([xw1, xw1a], axis=1), a_ref[:],
                     preferred_element_type=F32)


def _mid_body(p0_ref, p1_ref, pa0_ref, pa1_ref, b1_ref, w2_ref, a2_ref,
              z_ref, emb_ref, xw2a_ref, xw2b_ref, s2_ref, dout):
  p = p0_ref[:] + p1_ref[:]
  pa = pa0_ref[:] + pa1_ref[:]
  z = p[:, :dout] / (p[:, dout:dout + 1] + 1e-16) + b1_ref[:]
  za = pa[:, :dout] / (pa[:, dout:dout + 1] + 1e-16) + b1_ref[:]
  z_ref[:] = z
  emb_ref[:] = jnp.concatenate(
      [jnp.maximum(z, 0.0), jnp.maximum(za, 0.0)], axis=1)
  xw2 = jnp.dot(z, w2_ref[:], preferred_element_type=F32)
  xw2a_ref[:] = xw2[:, :TW]
  xw2b_ref[:] = xw2[:, TW:]
  s2_ref[:] = jnp.dot(xw2, a2_ref[:], preferred_element_type=F32)


def _readout_body(gn_ref, emb_ref, vs_ref, rs_ref):
  gn = gn_ref[:]
  vs_ref[:] = jnp.dot(gn, emb_ref[:], preferred_element_type=F32)
  rs_ref[:] = jnp.broadcast_to(jnp.sum(gn, axis=1, keepdims=True),
                               rs_ref.shape)


def _final_body(qa0_ref, qa1_ref, qb0_ref, qb1_ref, b2_ref, vs_ref, rs_ref,
                emb_ref, wbt_ref, h_ref, ret_ref, reta_ref, din, dout):
  qa = qa0_ref[:] + qa1_ref[:]
  qb = qb0_ref[:] + qb1_ref[:]
  ha = qa[:, :TW] / (qa[:, TW:TW + 1] + 1e-16)
  hb = qb[:, :TW] / (qb[:, TW:TW + 1] + 1e-16)
  h_ref[:] = jnp.concatenate([ha, hb], axis=1) + b2_ref[:]
  vs = vs_ref[:]
  rsum = rs_ref[:, 0:1]

  def norm_sig(x):
    nrm = jnp.sqrt(jnp.sum(x * x, axis=1, keepdims=True))
    xn = x / jnp.maximum(nrm, 1e-12)
    return 1.0 / (1.0 + jnp.exp(-xn))

  g = norm_sig(vs[:, :dout] / rsum)
  ga = norm_sig(vs[:, dout:2 * dout] / rsum)
  t = jnp.dot(g, wbt_ref[:], preferred_element_type=F32)
  ta = jnp.dot(ga, wbt_ref[:], preferred_element_type=F32)
  emb = emb_ref[:][:, :dout]
  emba = emb_ref[:][:, dout:2 * dout]
  ret_ref[:] = jnp.concatenate(
      [jnp.sum(emb * t, axis=1, keepdims=True),
       jnp.sum(emba * t, axis=1, keepdims=True)], axis=1)
  reta_ref[:] = jnp.concatenate(
      [jnp.sum(emba * ta, axis=1, keepdims=True),
       jnp.sum(emb * ta, axis=1, keepdims=True)], axis=1)


def _lrelu(x):
  return jnp.maximum(x, 0.2 * x)


def kernel(feat, feat_a, edge_index, graph_neigh, W1, a1_src, a1_dst, b1,
           W2, a2_src, a2_dst, b2, Wb, bb):
  n, din = feat.shape
  dout = W1.shape[1]
  e = edge_index.shape[1]

  rpad = -(-(n + 1) // (NS * K)) * (NS * K)
  edg = e + n
  ept = -(-edg // (NT * 2 * K)) * (2 * K)
  ep = ept * NT
  nchunk = ept // K

  loop = jnp.arange(n, dtype=I32)
  padv = jnp.full((ep - edg,), n, I32)
  src3 = jnp.concatenate([edge_index[0], loop, padv]).reshape(NT, nchunk, K)
  dst3 = jnp.concatenate([edge_index[1], loop, padv]).reshape(NT, nchunk, K)

  featp = jnp.pad(feat, ((0, rpad - n), (0, 0)))
  featap = jnp.pad(feat_a, ((0, rpad - n), (0, 0)))

  amat1 = jnp.zeros((2 * dout, 8), F32)
  amat1 = amat1.at[:dout, 0].set(a1_src).at[:dout, 1].set(a1_dst)
  amat1 = amat1.at[dout:, 2].set(a1_src).at[dout:, 3].set(a1_dst)
  bn = 1024
  g1 = rpad // bn
  xw1, xw1a, s1 = pl.pallas_call(
      _prep1_body,
      grid=(g1,),
      in_specs=[
          pl.BlockSpec((bn, din), lambda i: (i, 0)),
          pl.BlockSpec((bn, din), lambda i: (i, 0)),
          pl.BlockSpec((din, dout), lambda i: (0, 0)),
          pl.BlockSpec((2 * dout, 8), lambda i: (0, 0)),
      ],
      out_specs=[
          pl.BlockSpec((bn, dout), lambda i: (i, 0)),
          pl.BlockSpec((bn, dout), lambda i: (i, 0)),
          pl.BlockSpec((bn, 8), lambda i: (i, 0)),
      ],
      out_shape=[
          jax.ShapeDtypeStruct((rpad, dout), F32),
          jax.ShapeDtypeStruct((rpad, dout), F32),
          jax.ShapeDtypeStruct((rpad, 8), F32),
      ],
  )(featp, featap, W1, amat1)

  c1 = _lrelu(jnp.max(s1[:n, 0]) + jnp.max(s1[:n, 1]))
  c1a = _lrelu(jnp.max(s1[:n, 2]) + jnp.max(s1[:n, 3]))
  s1t = jnp.transpose(s1[:, :2]).reshape(2, rpad)
  s1at = jnp.transpose(s1[:, 2:4]).reshape(2, rpad)
  cv1 = jnp.full((16,), c1, F32)
  cv1a = jnp.full((16,), c1a, F32)

  conv = _make_sc_conv(nchunk, rpad)
  out1 = conv(src3, dst3, xw1, s1t, cv1)
  out1a = conv(src3, dst3, xw1a, s1at, cv1a)

  amat2 = jnp.zeros((din, 8), F32)
  amat2 = amat2.at[:, 0].set(a2_src).at[:, 1].set(a2_dst)
  zf, embb, xw2a, xw2b, s2 = pl.pallas_call(
      functools.partial(_mid_body, dout=dout),
      grid=(g1,),
      in_specs=[
          pl.BlockSpec((bn, SW), lambda i: (i, 0)),
          pl.BlockSpec((bn, SW), lambda i: (i, 0)),
          pl.BlockSpec((bn, SW), lambda i: (i, 0)),
          pl.BlockSpec((bn, SW), lambda i: (i, 0)),
          pl.BlockSpec((1, dout), lambda i: (0, 0)),
          pl.BlockSpec((dout, din), lambda i: (0, 0)),
          pl.BlockSpec((din, 8), lambda i: (0, 0)),
      ],
      out_specs=[
          pl.BlockSpec((bn, dout), lambda i: (i, 0)),
          pl.BlockSpec((bn, 2 * dout), lambda i: (i, 0)),
          pl.BlockSpec((bn, TW), lambda i: (i, 0)),
          pl.BlockSpec((bn, TW), lambda i: (i, 0)),
          pl.BlockSpec((bn, 8), lambda i: (i, 0)),
      ],
      out_shape=[
          jax.ShapeDtypeStruct((rpad, dout), F32),
          jax.ShapeDtypeStruct((rpad, 2 * dout), F32),
          jax.ShapeDtypeStruct((rpad, TW), F32),
          jax.ShapeDtypeStruct((rpad, TW), F32),
          jax.ShapeDtypeStruct((rpad, 8), F32),
      ],
  )(out1[0], out1[1], out1a[0], out1a[1], b1.reshape(1, dout), W2, amat2)

  c2 = _lrelu(jnp.max(s2[:n, 0]) + jnp.max(s2[:n, 1]))
  s2t = jnp.transpose(s2[:, :2]).reshape(2, rpad)
  cv2 = jnp.full((16,), c2, F32)

  out2a = conv(src3, dst3, xw2a, s2t, cv2)
  out2b = conv(src3, dst3, xw2b, s2t, cv2)

  bm = 400
  vs, rs = pl.pallas_call(
      _readout_body,
      grid=(n // bm,),
      in_specs=[
          pl.BlockSpec((bm, n), lambda i: (i, 0)),
          pl.BlockSpec((n, 2 * dout), lambda i: (0, 0)),
      ],
      out_specs=[
          pl.BlockSpec((bm, 2 * dout), lambda i: (i, 0)),
          pl.BlockSpec((bm, 8), lambda i: (i, 0)),
      ],
      out_shape=[
          jax.ShapeDtypeStruct((n, 2 * dout), F32),
          jax.ShapeDtypeStruct((n, 8), F32),
      ],
  )(graph_neigh, embb)

  bm2 = 1000
  h, ret, reta = pl.pallas_call(
      functools.partial(_final_body, din=din, dout=dout),
      grid=(n // bm2,),
      in_specs=[
          pl.BlockSpec((bm2, SW), lambda i: (i, 0)),
          pl.BlockSpec((bm2, SW), lambda i: (i, 0)),
          pl.BlockSpec((bm2, SW), lambda i: (i, 0)),
          pl.BlockSpec((bm2, SW), lambda i: (i, 0)),
          pl.BlockSpec((1, din), lambda i: (0, 0)),
          pl.BlockSpec((bm2, 2 * dout), lambda i: (i, 0)),
          pl.BlockSpec((bm2, 8), lambda i: (i, 0)),
          pl.BlockSpec((bm2, 2 * dout), lambda i: (i, 0)),
          pl.BlockSpec((dout, dout), lambda i: (0, 0)),
      ],
      out_specs=[
          pl.BlockSpec((bm2, din), lambda i: (i, 0)),
          pl.BlockSpec((bm2, 2), lambda i: (i, 0)),
          pl.BlockSpec((bm2, 2), lambda i: (i, 0)),
      ],
      out_shape=[
          jax.ShapeDtypeStruct((n, din), F32),
          jax.ShapeDtypeStruct((n, 2), F32),
          jax.ShapeDtypeStruct((n, 2), F32),
      ],
  )(out2a[0], out2a[1], out2b[0], out2b[1], b2.reshape(1, din), vs, rs,
    embb, Wb[0].T)

  hiden_emb = zf[:n]
  return (hiden_emb, h, ret + bb[0], reta + bb[0])
